# Initial kernel scaffold; baseline (speedup 1.0000x reference)
import jax
import jax.numpy as jnp
from jax.experimental import pallas as pl


def kernel(h, edge_index, eig, snorm_n, W1, b1, W2, b2, gamma, beta):
    raise NotImplementedError("write your pallas kernel here")


# speedup vs baseline: 3.2436x; 3.2436x over previous
import functools

import jax
import jax.numpy as jnp
from jax import lax
from jax.experimental import pallas as pl
from jax.experimental.pallas import tpu as pltpu
from jax.experimental.pallas import tpu_sc as plsc

EPS = 1e-8
BN_EPS = 1e-5


def _mm_body(h_ref, w_ref, o_ref):
    o_ref[...] = jnp.dot(h_ref[...], w_ref[...],
                         preferred_element_type=jnp.float32)


def _pretrans(h, wcat, row_block):
    n, d = h.shape
    grid = n // row_block
    return pl.pallas_call(
        _mm_body,
        grid=(grid,),
        in_specs=[
            pl.BlockSpec((row_block, d), lambda i: (i, 0)),
            pl.BlockSpec((d, 2 * d), lambda i: (0, 0)),
        ],
        out_specs=pl.BlockSpec((row_block, 2 * d), lambda i: (i, 0)),
        out_shape=jax.ShapeDtypeStruct((n, 2 * d), jnp.float32),
    )(h, wcat)


NT = 16
CHUNK = 32
CHUNK2 = 80
DH = 64

_SC_MESH = plsc.VectorSubcoreMesh(core_axis_name="c", subcore_axis_name="s")
_SC_PARAMS = pltpu.CompilerParams(use_tc_tiling_on_sc=False)


def _make_sc_wide(n, e):
    edges_per_tile = e // NT
    n_chunks = edges_per_tile // CHUNK
    r0 = (n // NT) // 8 * 8
    r15 = n - (NT - 1) * r0
    f32 = jnp.float32

    @functools.partial(
        pl.kernel,
        mesh=_SC_MESH,
        compiler_params=_SC_PARAMS,
        out_type=(
            jax.ShapeDtypeStruct((2, n, DH), f32),
            jax.ShapeDtypeStruct((2, n, DH), f32),
            jax.ShapeDtypeStruct((2, n, DH), f32),
        ),
        scratch_types=[
            pltpu.VMEM((CHUNK,), jnp.int32),
            pltpu.VMEM((CHUNK,), jnp.int32),
            pltpu.VMEM((CHUNK,), f32),
            pltpu.VMEM((CHUNK, DH), f32),
            pltpu.VMEM((CHUNK, DH), f32),
            pltpu.VMEM((CHUNK, DH), f32),
            pltpu.VMEM_SHARED((n, DH), f32),
            pltpu.VMEM_SHARED((n, DH), f32),
            pltpu.VMEM_SHARED((n, DH), f32),
            pltpu.SemaphoreType.DMA,
        ],
    )
    def sc_wide(a2_hbm, src2_hbm, dst_hbm, eig_hbm, zeros_hbm,
                t0_hbm, t1_hbm, t2_hbm,
                srcv, dstv, eigv, g, g1, g2,
                acc0, acc1, acc2, sem):
        c = lax.axis_index("c")
        s = lax.axis_index("s")

        row0 = pl.multiple_of(s * r0, 8)

        @pl.when(s < NT - 1)
        def _():
            sl = pl.ds(row0, r0)
            pltpu.sync_copy(zeros_hbm.at[sl], acc0.at[sl])
            pltpu.sync_copy(zeros_hbm.at[sl], acc1.at[sl])
            pltpu.sync_copy(zeros_hbm.at[sl], acc2.at[sl])

        @pl.when(s == NT - 1)
        def _():
            sl = pl.ds((NT - 1) * r0, r15)
            pltpu.sync_copy(zeros_hbm.at[sl], acc0.at[sl])
            pltpu.sync_copy(zeros_hbm.at[sl], acc1.at[sl])
            pltpu.sync_copy(zeros_hbm.at[sl], acc2.at[sl])

        plsc.subcore_barrier()

        tile_base = s * edges_per_tile

        def _chunk(ci, _):
            base = tile_base + ci * CHUNK
            src_off = pl.multiple_of(c * e + base, 8)
            pltpu.sync_copy(src2_hbm.at[pl.ds(src_off, CHUNK)], srcv)
            pltpu.sync_copy(dst_hbm.at[pl.ds(base, CHUNK)], dstv)
            pltpu.sync_copy(eig_hbm.at[pl.ds(base, CHUNK)], eigv)
            pltpu.async_copy(a2_hbm.at[srcv], g, sem).wait()

            def _grp(i, _):
                ev16 = eigv[pl.ds(16 * i, 16)]
                ea16 = jnp.abs(ev16)
                for j in range(16):
                    es = ev16[j]
                    ea = ea16[j]
                    k = 16 * i + j
                    for fb in range(DH // 16):
                        sl = pl.ds(16 * fb, 16)
                        gv = g[k, sl]
                        g1[k, sl] = gv * ea
                        g2[k, sl] = gv * es
                return 0

            lax.fori_loop(0, CHUNK // 16, _grp, 0)

            pltpu.sync_copy(g, acc0.at[dstv], add=True)
            pltpu.sync_copy(g1, acc1.at[dstv], add=True)
            pltpu.sync_copy(g2, acc2.at[dstv], add=True)
            return 0

        lax.fori_loop(0, n_chunks, _chunk, 0)

        plsc.subcore_barrier()

        @pl.when(s < NT - 1)
        def _():
            sl = pl.ds(row0, r0)
            pltpu.sync_copy(acc0.at[sl], t0_hbm.at[c, sl])
            pltpu.sync_copy(acc1.at[sl], t1_hbm.at[c, sl])
            pltpu.sync_copy(acc2.at[sl], t2_hbm.at[c, sl])

        @pl.when(s == NT - 1)
        def _():
            sl = pl.ds((NT - 1) * r0, r15)
            pltpu.sync_copy(acc0.at[sl], t0_hbm.at[c, sl])
            pltpu.sync_copy(acc1.at[sl], t1_hbm.at[c, sl])
            pltpu.sync_copy(acc2.at[sl], t2_hbm.at[c, sl])

    return sc_wide


def _make_sc_scalar(n, e):
    ehalf = e // 2
    ept = ehalf // NT
    n_chunks = ept // CHUNK2
    r0 = (n // NT) // 8 * 8
    r15 = n - (NT - 1) * r0
    f32 = jnp.float32

    @functools.partial(
        pl.kernel,
        mesh=_SC_MESH,
        compiler_params=_SC_PARAMS,
        out_type=jax.ShapeDtypeStruct((2, n, 16), f32),
        scratch_types=[
            pltpu.VMEM((CHUNK2,), jnp.int32),
            pltpu.VMEM((CHUNK2,), f32),
            pltpu.VMEM((CHUNK2, 16), f32),
            pltpu.VMEM_SHARED((n, 16), f32),
        ],
    )
    def sc_scalar(dst_hbm, eig_hbm, zeros_hbm, s_hbm, dstv, eigv, scl, accs):
        c = lax.axis_index("c")
        s = lax.axis_index("s")

        row0 = pl.multiple_of(s * r0, 8)

        @pl.when(s < NT - 1)
        def _():
            sl = pl.ds(row0, r0)
            pltpu.sync_copy(zeros_hbm.at[sl], accs.at[sl])

        @pl.when(s == NT - 1)
        def _():
            sl = pl.ds((NT - 1) * r0, r15)
            pltpu.sync_copy(zeros_hbm.at[sl], accs.at[sl])

        plsc.subcore_barrier()

        lane = lax.iota(jnp.int32, 16)
        is0 = lane == 0
        is1 = lane == 1
        is2 = lane == 2
        tile_base = c * ehalf + s * ept

        def _chunk(ci, _):
            base = pl.multiple_of(tile_base + ci * CHUNK2, 8)
            pltpu.sync_copy(dst_hbm.at[pl.ds(base, CHUNK2)], dstv)
            pltpu.sync_copy(eig_hbm.at[pl.ds(base, CHUNK2)], eigv)

            def _grp(i, _):
                ev16 = eigv[pl.ds(16 * i, 16)]
                ea16 = jnp.abs(ev16)
                for j in range(16):
                    row = jnp.where(is0, jnp.float32(1.0),
                                    jnp.where(is1, ea16[j],
                                              jnp.where(is2, ev16[j],
                                                        jnp.float32(0.0))))
                    scl[16 * i + j, pl.ds(0, 16)] = row
                return 0

            lax.fori_loop(0, CHUNK2 // 16, _grp, 0)
            pltpu.sync_copy(scl, accs.at[dstv], add=True)
            return 0

        lax.fori_loop(0, n_chunks, _chunk, 0)

        plsc.subcore_barrier()

        @pl.when(s < NT - 1)
        def _():
            sl = pl.ds(row0, r0)
            pltpu.sync_copy(accs.at[sl], s_hbm.at[c, sl])

        @pl.when(s == NT - 1)
        def _():
            sl = pl.ds((NT - 1) * r0, r15)
            pltpu.sync_copy(accs.at[sl], s_hbm.at[c, sl])

    return sc_scalar


def _post_body(h_ref, b_ref, t0_ref, t1_ref, t2_ref, sa_ref, sb_ref, sn_ref,
               w2_ref, b1_ref, b2_ref, y_ref, s1_ref, s2_ref):
    i = pl.program_id(0)
    h = h_ref[...]
    bb = b_ref[...] + b1_ref[...]
    sc = sa_ref[...] + sb_ref[...]
    deg = sc[:, 0:1]
    asum = sc[:, 1:2]
    esum = sc[:, 2:3]
    degc = jnp.maximum(deg, 1.0)
    r = 1.0 / (asum + EPS)
    agg_mean = (t0_ref[...] + deg * bb) / degc
    agg_av = (t1_ref[...] + asum * bb) * r
    agg_dx = jnp.abs((t2_ref[...] + esum * bb) * r - (esum * r) * h)
    x = jnp.concatenate([h, agg_mean, agg_av, agg_dx], axis=1)
    y = jnp.dot(x, w2_ref[...], preferred_element_type=jnp.float32)
    y = (y + b2_ref[...]) * sn_ref[...]
    y_ref[...] = y

    @pl.when(i == 0)
    def _():
        s1_ref[...] = jnp.zeros_like(s1_ref)
        s2_ref[...] = jnp.zeros_like(s2_ref)

    s1_ref[...] += jnp.sum(y, axis=0, keepdims=True)
    s2_ref[...] += jnp.sum(y * y, axis=0, keepdims=True)


def _posttrans(h, b, t0, t1, t2, scal_a, scal_b, snorm, w2, b1r, b2r,
               row_block):
    n, d = h.shape
    grid = n // row_block
    rb = row_block
    return pl.pallas_call(
        _post_body,
        grid=(grid,),
        in_specs=[
            pl.BlockSpec((rb, d), lambda i: (i, 0)),
            pl.BlockSpec((rb, d), lambda i: (i, 0)),
            pl.BlockSpec((rb, d), lambda i: (i, 0)),
            pl.BlockSpec((rb, d), lambda i: (i, 0)),
            pl.BlockSpec((rb, d), lambda i: (i, 0)),
            pl.BlockSpec((rb, 16), lambda i: (i, 0)),
            pl.BlockSpec((rb, 16), lambda i: (i, 0)),
            pl.BlockSpec((rb, 1), lambda i: (i, 0)),
            pl.BlockSpec((4 * d, d), lambda i: (0, 0)),
            pl.BlockSpec((1, d), lambda i: (0, 0)),
            pl.BlockSpec((1, d), lambda i: (0, 0)),
        ],
        out_specs=[
            pl.BlockSpec((rb, d), lambda i: (i, 0)),
            pl.BlockSpec((1, d), lambda i: (0, 0)),
            pl.BlockSpec((1, d), lambda i: (0, 0)),
        ],
        out_shape=[
            jax.ShapeDtypeStruct((n, d), jnp.float32),
            jax.ShapeDtypeStruct((1, d), jnp.float32),
            jax.ShapeDtypeStruct((1, d), jnp.float32),
        ],
    )(h, b, t0, t1, t2, scal_a, scal_b, snorm, w2, b1r, b2r)


def _bn_body(y_ref, h_ref, s1_ref, s2_ref, g_ref, be_ref, n_ref, o_ref):
    n = n_ref[0]
    mu = s1_ref[...] / n
    var = s2_ref[...] / n - mu * mu
    inv = lax.rsqrt(var + BN_EPS)
    yn = (y_ref[...] - mu) * (inv * g_ref[...]) + be_ref[...]
    o_ref[...] = h_ref[...] + jnp.maximum(yn, 0.0)


def _bn_apply(y, h, s1, s2, gr, br, row_block):
    n, d = h.shape
    grid = n // row_block
    rb = row_block
    nvec = jnp.full((1,), float(n), dtype=jnp.float32)
    return pl.pallas_call(
        _bn_body,
        grid=(grid,),
        in_specs=[
            pl.BlockSpec((rb, d), lambda i: (i, 0)),
            pl.BlockSpec((rb, d), lambda i: (i, 0)),
            pl.BlockSpec((1, d), lambda i: (0, 0)),
            pl.BlockSpec((1, d), lambda i: (0, 0)),
            pl.BlockSpec((1, d), lambda i: (0, 0)),
            pl.BlockSpec((1, d), lambda i: (0, 0)),
            pl.BlockSpec(memory_space=pltpu.SMEM),
        ],
        out_specs=pl.BlockSpec((rb, d), lambda i: (i, 0)),
        out_shape=jax.ShapeDtypeStruct((n, d), jnp.float32),
    )(y, h, s1, s2, gr, br, nvec)


def kernel(h, edge_index, eig, snorm_n, W1, b1, W2, b2, gamma, beta):
    n, d = h.shape
    e = edge_index.shape[1]

    wcat = jnp.concatenate([W1[:d], W1[d:]], axis=1)
    p = _pretrans(h, wcat, row_block=1000)
    a = p[:, :d]
    b = p[:, d:]
    a2 = jnp.concatenate([a[:, :DH], a[:, DH:]], axis=0)

    src = edge_index[0]
    dst = edge_index[1]
    src2 = jnp.concatenate([src, src + n])
    ev = eig[:, 0]

    zw = jnp.zeros((n, DH), jnp.float32)
    zs = jnp.zeros((n, 16), jnp.float32)
    t0h, t1h, t2h = _make_sc_wide(n, e)(a2, src2, dst, ev, zw)
    sh = _make_sc_scalar(n, e)(dst, ev, zs)
    t0 = jnp.concatenate([t0h[0], t0h[1]], axis=1)
    t1 = jnp.concatenate([t1h[0], t1h[1]], axis=1)
    t2 = jnp.concatenate([t2h[0], t2h[1]], axis=1)

    b1r = b1.reshape(1, d)
    b2r = b2.reshape(1, d)
    y, s1, s2 = _posttrans(h, b, t0, t1, t2, sh[0], sh[1], snorm_n, W2,
                           b1r, b2r, row_block=1000)
    return _bn_apply(y, h, s1, s2, gamma.reshape(1, d), beta.reshape(1, d),
                     row_block=1000)


# speedup vs baseline: 6.7374x; 2.0772x over previous
import functools

import jax
import jax.numpy as jnp
from jax import lax
from jax.experimental import pallas as pl
from jax.experimental.pallas import tpu as pltpu
from jax.experimental.pallas import tpu_sc as plsc

EPS = 1e-8
BN_EPS = 1e-5


def _mm_body(h_ref, w_ref, o_ref):
    o_ref[...] = jnp.dot(h_ref[...], w_ref[...],
                         preferred_element_type=jnp.float32)


def _pretrans(h, wcat, row_block):
    n, d = h.shape
    grid = n // row_block
    return pl.pallas_call(
        _mm_body,
        grid=(grid,),
        in_specs=[
            pl.BlockSpec((row_block, d), lambda i: (i, 0)),
            pl.BlockSpec((d, 2 * d), lambda i: (0, 0)),
        ],
        out_specs=pl.BlockSpec((row_block, 2 * d), lambda i: (i, 0)),
        out_shape=jax.ShapeDtypeStruct((n, 2 * d), jnp.float32),
    )(h, wcat)


NT = 16
CHUNK = 16
NBUF = 3
CHUNK2 = 80
DH = 64

_SC_MESH = plsc.VectorSubcoreMesh(core_axis_name="c", subcore_axis_name="s")
_SC_PARAMS = pltpu.CompilerParams(use_tc_tiling_on_sc=False)


def _make_sc_wide(n, e):
    edges_per_tile = e // NT
    n_chunks = edges_per_tile // CHUNK
    r0 = (n // NT) // 8 * 8
    r15 = n - (NT - 1) * r0
    f32 = jnp.float32

    @functools.partial(
        pl.kernel,
        mesh=_SC_MESH,
        compiler_params=_SC_PARAMS,
        out_type=(
            jax.ShapeDtypeStruct((2, n, DH), f32),
            jax.ShapeDtypeStruct((2, n, DH), f32),
            jax.ShapeDtypeStruct((2, n, DH), f32),
        ),
        scratch_types=[
            pltpu.VMEM((NBUF, CHUNK), jnp.int32),
            pltpu.VMEM((NBUF, CHUNK), jnp.int32),
            pltpu.VMEM((NBUF, CHUNK), f32),
            pltpu.VMEM((NBUF, CHUNK, DH), f32),
            pltpu.VMEM((NBUF, CHUNK, DH), f32),
            pltpu.VMEM((NBUF, CHUNK, DH), f32),
            pltpu.VMEM_SHARED((n, DH), f32),
            pltpu.VMEM_SHARED((n, DH), f32),
            pltpu.VMEM_SHARED((n, DH), f32),
            pltpu.SemaphoreType.DMA((NBUF,)),
            pltpu.SemaphoreType.DMA((NBUF,)),
            pltpu.SemaphoreType.DMA((NBUF,)),
        ],
    )
    def sc_wide(a2_hbm, src2_hbm, dst_hbm, eig_hbm, zeros_hbm,
                t0_hbm, t1_hbm, t2_hbm,
                srcv, dstv, eigv, g, g1, g2,
                acc0, acc1, acc2, sem_a, sem_b, sem_c):
        c = lax.axis_index("c")
        s = lax.axis_index("s")

        row0 = pl.multiple_of(s * r0, 8)

        @pl.when(s < NT - 1)
        def _():
            sl = pl.ds(row0, r0)
            pltpu.sync_copy(zeros_hbm.at[sl], acc0.at[sl])
            pltpu.sync_copy(zeros_hbm.at[sl], acc1.at[sl])
            pltpu.sync_copy(zeros_hbm.at[sl], acc2.at[sl])

        @pl.when(s == NT - 1)
        def _():
            sl = pl.ds((NT - 1) * r0, r15)
            pltpu.sync_copy(zeros_hbm.at[sl], acc0.at[sl])
            pltpu.sync_copy(zeros_hbm.at[sl], acc1.at[sl])
            pltpu.sync_copy(zeros_hbm.at[sl], acc2.at[sl])

        plsc.subcore_barrier()

        tile_base = s * edges_per_tile

        def _load(j):
            slot = lax.rem(j, NBUF)
            base = tile_base + j * CHUNK
            src_off = pl.multiple_of(c * e + base, 8)
            pltpu.async_copy(src2_hbm.at[pl.ds(src_off, CHUNK)],
                             srcv.at[slot], sem_a.at[slot])
            pltpu.async_copy(dst_hbm.at[pl.ds(base, CHUNK)],
                             dstv.at[slot], sem_a.at[slot])
            pltpu.async_copy(eig_hbm.at[pl.ds(base, CHUNK)],
                             eigv.at[slot], sem_a.at[slot])

        def _issue_gather(j):
            slot = lax.rem(j, NBUF)
            pltpu.make_async_copy(
                src2_hbm.at[pl.ds(0, CHUNK)], srcv.at[slot],
                sem_a.at[slot]).wait()
            pltpu.make_async_copy(
                dst_hbm.at[pl.ds(0, CHUNK)], dstv.at[slot],
                sem_a.at[slot]).wait()
            pltpu.make_async_copy(
                eig_hbm.at[pl.ds(0, CHUNK)], eigv.at[slot],
                sem_a.at[slot]).wait()
            pltpu.async_copy(a2_hbm.at[srcv.at[slot]], g.at[slot],
                             sem_b.at[slot])

        def _wait_scatters(slot):
            for buf in (g, g1, g2):
                pltpu.make_async_copy(
                    zeros_hbm.at[pl.ds(0, CHUNK)], buf.at[slot],
                    sem_c.at[slot]).wait()

        def _compute_scatter(j):
            slot = lax.rem(j, NBUF)
            pltpu.make_async_copy(
                zeros_hbm.at[pl.ds(0, CHUNK)], g.at[slot],
                sem_b.at[slot]).wait()

            def _grp(i, _):
                ev16 = eigv[slot, pl.ds(16 * i, 16)]
                ea16 = jnp.abs(ev16)
                for jj in range(16):
                    es = ev16[jj]
                    ea = ea16[jj]
                    k = 16 * i + jj
                    for fb in range(DH // 16):
                        sl = pl.ds(16 * fb, 16)
                        gv = g[slot, k, sl]
                        g1[slot, k, sl] = gv * ea
                        g2[slot, k, sl] = gv * es
                return 0

            lax.fori_loop(0, CHUNK // 16, _grp, 0)

            pltpu.async_copy(g.at[slot], acc0.at[dstv.at[slot]],
                             sem_c.at[slot], add=True)
            pltpu.async_copy(g1.at[slot], acc1.at[dstv.at[slot]],
                             sem_c.at[slot], add=True)
            pltpu.async_copy(g2.at[slot], acc2.at[dstv.at[slot]],
                             sem_c.at[slot], add=True)

        _load(0)
        _load(1)
        _issue_gather(0)

        def _body(ci, _):
            @pl.when(ci + 1 < n_chunks)
            def _():
                _issue_gather(ci + 1)

            @pl.when(ci + 2 < n_chunks)
            def _():
                @pl.when(ci >= NBUF - 2)
                def _():
                    _wait_scatters(lax.rem(ci + 2, NBUF))
                _load(ci + 2)

            _compute_scatter(ci)
            return 0

        lax.fori_loop(0, n_chunks, _body, 0)

        for j in range(max(0, n_chunks - NBUF), n_chunks):
            _wait_scatters(j % NBUF)

        plsc.subcore_barrier()

        @pl.when(s < NT - 1)
        def _():
            sl = pl.ds(row0, r0)
            pltpu.sync_copy(acc0.at[sl], t0_hbm.at[c, sl])
            pltpu.sync_copy(acc1.at[sl], t1_hbm.at[c, sl])
            pltpu.sync_copy(acc2.at[sl], t2_hbm.at[c, sl])

        @pl.when(s == NT - 1)
        def _():
            sl = pl.ds((NT - 1) * r0, r15)
            pltpu.sync_copy(acc0.at[sl], t0_hbm.at[c, sl])
            pltpu.sync_copy(acc1.at[sl], t1_hbm.at[c, sl])
            pltpu.sync_copy(acc2.at[sl], t2_hbm.at[c, sl])

    return sc_wide


def _make_sc_scalar(n, e):
    ehalf = e // 2
    ept = ehalf // NT
    n_chunks = ept // CHUNK2
    r0 = (n // NT) // 8 * 8
    r15 = n - (NT - 1) * r0
    f32 = jnp.float32

    @functools.partial(
        pl.kernel,
        mesh=_SC_MESH,
        compiler_params=_SC_PARAMS,
        out_type=jax.ShapeDtypeStruct((2, n, 16), f32),
        scratch_types=[
            pltpu.VMEM((CHUNK2,), jnp.int32),
            pltpu.VMEM((CHUNK2,), f32),
            pltpu.VMEM((CHUNK2, 16), f32),
            pltpu.VMEM_SHARED((n, 16), f32),
        ],
    )
    def sc_scalar(dst_hbm, eig_hbm, zeros_hbm, s_hbm, dstv, eigv, scl, accs):
        c = lax.axis_index("c")
        s = lax.axis_index("s")

        row0 = pl.multiple_of(s * r0, 8)

        @pl.when(s < NT - 1)
        def _():
            sl = pl.ds(row0, r0)
            pltpu.sync_copy(zeros_hbm.at[sl], accs.at[sl])

        @pl.when(s == NT - 1)
        def _():
            sl = pl.ds((NT - 1) * r0, r15)
            pltpu.sync_copy(zeros_hbm.at[sl], accs.at[sl])

        plsc.subcore_barrier()

        lane = lax.iota(jnp.int32, 16)
        is0 = lane == 0
        is1 = lane == 1
        is2 = lane == 2
        tile_base = c * ehalf + s * ept

        def _chunk(ci, _):
            base = pl.multiple_of(tile_base + ci * CHUNK2, 8)
            pltpu.sync_copy(dst_hbm.at[pl.ds(base, CHUNK2)], dstv)
            pltpu.sync_copy(eig_hbm.at[pl.ds(base, CHUNK2)], eigv)

            def _grp(i, _):
                ev16 = eigv[pl.ds(16 * i, 16)]
                ea16 = jnp.abs(ev16)
                for j in range(16):
                    row = jnp.where(is0, jnp.float32(1.0),
                                    jnp.where(is1, ea16[j],
                                              jnp.where(is2, ev16[j],
                                                        jnp.float32(0.0))))
                    scl[16 * i + j, pl.ds(0, 16)] = row
                return 0

            lax.fori_loop(0, CHUNK2 // 16, _grp, 0)
            pltpu.sync_copy(scl, accs.at[dstv], add=True)
            return 0

        lax.fori_loop(0, n_chunks, _chunk, 0)

        plsc.subcore_barrier()

        @pl.when(s < NT - 1)
        def _():
            sl = pl.ds(row0, r0)
            pltpu.sync_copy(accs.at[sl], s_hbm.at[c, sl])

        @pl.when(s == NT - 1)
        def _():
            sl = pl.ds((NT - 1) * r0, r15)
            pltpu.sync_copy(accs.at[sl], s_hbm.at[c, sl])

    return sc_scalar


def _post_body(h_ref, b_ref, t0_ref, t1_ref, t2_ref, sa_ref, sb_ref, sn_ref,
               w2_ref, b1_ref, b2_ref, y_ref, s1_ref, s2_ref):
    i = pl.program_id(0)
    h = h_ref[...]
    bb = b_ref[...] + b1_ref[...]
    sc = sa_ref[...] + sb_ref[...]
    deg = sc[:, 0:1]
    asum = sc[:, 1:2]
    esum = sc[:, 2:3]
    degc = jnp.maximum(deg, 1.0)
    r = 1.0 / (asum + EPS)
    agg_mean = (t0_ref[...] + deg * bb) / degc
    agg_av = (t1_ref[...] + asum * bb) * r
    agg_dx = jnp.abs((t2_ref[...] + esum * bb) * r - (esum * r) * h)
    x = jnp.concatenate([h, agg_mean, agg_av, agg_dx], axis=1)
    y = jnp.dot(x, w2_ref[...], preferred_element_type=jnp.float32)
    y = (y + b2_ref[...]) * sn_ref[...]
    y_ref[...] = y

    @pl.when(i == 0)
    def _():
        s1_ref[...] = jnp.zeros_like(s1_ref)
        s2_ref[...] = jnp.zeros_like(s2_ref)

    s1_ref[...] += jnp.sum(y, axis=0, keepdims=True)
    s2_ref[...] += jnp.sum(y * y, axis=0, keepdims=True)


def _posttrans(h, b, t0, t1, t2, scal_a, scal_b, snorm, w2, b1r, b2r,
               row_block):
    n, d = h.shape
    grid = n // row_block
    rb = row_block
    return pl.pallas_call(
        _post_body,
        grid=(grid,),
        in_specs=[
            pl.BlockSpec((rb, d), lambda i: (i, 0)),
            pl.BlockSpec((rb, d), lambda i: (i, 0)),
            pl.BlockSpec((rb, d), lambda i: (i, 0)),
            pl.BlockSpec((rb, d), lambda i: (i, 0)),
            pl.BlockSpec((rb, d), lambda i: (i, 0)),
            pl.BlockSpec((rb, 16), lambda i: (i, 0)),
            pl.BlockSpec((rb, 16), lambda i: (i, 0)),
            pl.BlockSpec((rb, 1), lambda i: (i, 0)),
            pl.BlockSpec((4 * d, d), lambda i: (0, 0)),
            pl.BlockSpec((1, d), lambda i: (0, 0)),
            pl.BlockSpec((1, d), lambda i: (0, 0)),
        ],
        out_specs=[
            pl.BlockSpec((rb, d), lambda i: (i, 0)),
            pl.BlockSpec((1, d), lambda i: (0, 0)),
            pl.BlockSpec((1, d), lambda i: (0, 0)),
        ],
        out_shape=[
            jax.ShapeDtypeStruct((n, d), jnp.float32),
            jax.ShapeDtypeStruct((1, d), jnp.float32),
            jax.ShapeDtypeStruct((1, d), jnp.float32),
        ],
    )(h, b, t0, t1, t2, scal_a, scal_b, snorm, w2, b1r, b2r)


def _bn_body(y_ref, h_ref, s1_ref, s2_ref, g_ref, be_ref, n_ref, o_ref):
    n = n_ref[0]
    mu = s1_ref[...] / n
    var = s2_ref[...] / n - mu * mu
    inv = lax.rsqrt(var + BN_EPS)
    yn = (y_ref[...] - mu) * (inv * g_ref[...]) + be_ref[...]
    o_ref[...] = h_ref[...] + jnp.maximum(yn, 0.0)


def _bn_apply(y, h, s1, s2, gr, br, row_block):
    n, d = h.shape
    grid = n // row_block
    rb = row_block
    nvec = jnp.full((1,), float(n), dtype=jnp.float32)
    return pl.pallas_call(
        _bn_body,
        grid=(grid,),
        in_specs=[
            pl.BlockSpec((rb, d), lambda i: (i, 0)),
            pl.BlockSpec((rb, d), lambda i: (i, 0)),
            pl.BlockSpec((1, d), lambda i: (0, 0)),
            pl.BlockSpec((1, d), lambda i: (0, 0)),
            pl.BlockSpec((1, d), lambda i: (0, 0)),
            pl.BlockSpec((1, d), lambda i: (0, 0)),
            pl.BlockSpec(memory_space=pltpu.SMEM),
        ],
        out_specs=pl.BlockSpec((rb, d), lambda i: (i, 0)),
        out_shape=jax.ShapeDtypeStruct((n, d), jnp.float32),
    )(y, h, s1, s2, gr, br, nvec)


def kernel(h, edge_index, eig, snorm_n, W1, b1, W2, b2, gamma, beta):
    n, d = h.shape
    e = edge_index.shape[1]

    wcat = jnp.concatenate([W1[:d], W1[d:]], axis=1)
    p = _pretrans(h, wcat, row_block=1000)
    a = p[:, :d]
    b = p[:, d:]
    a2 = jnp.concatenate([a[:, :DH], a[:, DH:]], axis=0)

    src = edge_index[0]
    dst = edge_index[1]
    src2 = jnp.concatenate([src, src + n])
    ev = eig[:, 0]

    zw = jnp.zeros((n, DH), jnp.float32)
    zs = jnp.zeros((n, 16), jnp.float32)
    t0h, t1h, t2h = _make_sc_wide(n, e)(a2, src2, dst, ev, zw)
    sh = _make_sc_scalar(n, e)(dst, ev, zs)
    t0 = jnp.concatenate([t0h[0], t0h[1]], axis=1)
    t1 = jnp.concatenate([t1h[0], t1h[1]], axis=1)
    t2 = jnp.concatenate([t2h[0], t2h[1]], axis=1)

    b1r = b1.reshape(1, d)
    b2r = b2.reshape(1, d)
    y, s1, s2 = _posttrans(h, b, t0, t1, t2, sh[0], sh[1], snorm_n, W2,
                           b1r, b2r, row_block=1000)
    return _bn_apply(y, h, s1, s2, gamma.reshape(1, d), beta.reshape(1, d),
                     row_block=1000)


# speedup vs baseline: 8.1898x; 1.2156x over previous
import functools

import jax
import jax.numpy as jnp
from jax import lax
from jax.experimental import pallas as pl
from jax.experimental.pallas import tpu as pltpu
from jax.experimental.pallas import tpu_sc as plsc

EPS = 1e-8
BN_EPS = 1e-5


def _mm_body(h_ref, w_ref, o_ref):
    o_ref[...] = jnp.dot(h_ref[...], w_ref[...],
                         preferred_element_type=jnp.float32)


def _pretrans(h, wcat, row_block):
    n, d = h.shape
    grid = n // row_block
    return pl.pallas_call(
        _mm_body,
        grid=(grid,),
        in_specs=[
            pl.BlockSpec((row_block, d), lambda i: (i, 0)),
            pl.BlockSpec((d, 2 * d), lambda i: (0, 0)),
        ],
        out_specs=pl.BlockSpec((row_block, 2 * d), lambda i: (i, 0)),
        out_shape=jax.ShapeDtypeStruct((n, 2 * d), jnp.float32),
    )(h, wcat)


NT = 16
CHUNK = 80
NBUF = 4
CHUNK2 = 80
NBUF2 = 4
DH = 32

_SC_MESH = plsc.VectorSubcoreMesh(core_axis_name="c", subcore_axis_name="s")
_SC_PARAMS = pltpu.CompilerParams(use_tc_tiling_on_sc=False)


def _make_sc_wide(n, e):
    edges_per_tile = e // NT
    n_chunks = edges_per_tile // CHUNK
    r0 = (n // NT) // 8 * 8
    r15 = n - (NT - 1) * r0
    f32 = jnp.float32

    @functools.partial(
        pl.kernel,
        mesh=_SC_MESH,
        compiler_params=_SC_PARAMS,
        out_type=(
            jax.ShapeDtypeStruct((4, n, DH), f32),
            jax.ShapeDtypeStruct((4, n, DH), f32),
            jax.ShapeDtypeStruct((4, n, DH), f32),
        ),
        scratch_types=[
            pltpu.VMEM((NBUF, CHUNK), jnp.int32),
            pltpu.VMEM((NBUF, CHUNK), jnp.int32),
            pltpu.VMEM((NBUF, CHUNK), f32),
            pltpu.VMEM((NBUF, CHUNK, DH), f32),
            pltpu.VMEM((NBUF, CHUNK, DH), f32),
            pltpu.VMEM((NBUF, CHUNK, DH), f32),
            pltpu.VMEM_SHARED((n, DH), f32),
            pltpu.VMEM_SHARED((n, DH), f32),
            pltpu.VMEM_SHARED((n, DH), f32),
            pltpu.SemaphoreType.DMA((NBUF,)),
            pltpu.SemaphoreType.DMA((NBUF,)),
            pltpu.SemaphoreType.DMA((NBUF,)),
        ],
    )
    def sc_wide(a4_hbm, src4_hbm, dst_hbm, eig_hbm, zeros_hbm,
                t0_hbm, t1_hbm, t2_hbm,
                srcv, dstv, eigv, g, g1, g2,
                acc0, acc1, acc2, sem_a, sem_b, sem_c):
        c = lax.axis_index("c")
        s = lax.axis_index("s")
        row0 = pl.multiple_of(s * r0, 8)
        tile_base = s * edges_per_tile

        def _zero_accs():
            @pl.when(s < NT - 1)
            def _():
                sl = pl.ds(row0, r0)
                pltpu.sync_copy(zeros_hbm.at[sl], acc0.at[sl])
                pltpu.sync_copy(zeros_hbm.at[sl], acc1.at[sl])
                pltpu.sync_copy(zeros_hbm.at[sl], acc2.at[sl])

            @pl.when(s == NT - 1)
            def _():
                sl = pl.ds((NT - 1) * r0, r15)
                pltpu.sync_copy(zeros_hbm.at[sl], acc0.at[sl])
                pltpu.sync_copy(zeros_hbm.at[sl], acc1.at[sl])
                pltpu.sync_copy(zeros_hbm.at[sl], acc2.at[sl])

        def _copy_out(q):
            @pl.when(s < NT - 1)
            def _():
                sl = pl.ds(row0, r0)
                pltpu.sync_copy(acc0.at[sl], t0_hbm.at[q, sl])
                pltpu.sync_copy(acc1.at[sl], t1_hbm.at[q, sl])
                pltpu.sync_copy(acc2.at[sl], t2_hbm.at[q, sl])

            @pl.when(s == NT - 1)
            def _():
                sl = pl.ds((NT - 1) * r0, r15)
                pltpu.sync_copy(acc0.at[sl], t0_hbm.at[q, sl])
                pltpu.sync_copy(acc1.at[sl], t1_hbm.at[q, sl])
                pltpu.sync_copy(acc2.at[sl], t2_hbm.at[q, sl])

        def _run_phase(q):
            def _load(j):
                slot = lax.rem(j, NBUF)
                base = tile_base + j * CHUNK
                src_off = pl.multiple_of(q * e + base, 8)
                pltpu.async_copy(src4_hbm.at[pl.ds(src_off, CHUNK)],
                                 srcv.at[slot], sem_a.at[slot])
                pltpu.async_copy(dst_hbm.at[pl.ds(base, CHUNK)],
                                 dstv.at[slot], sem_a.at[slot])
                pltpu.async_copy(eig_hbm.at[pl.ds(base, CHUNK)],
                                 eigv.at[slot], sem_a.at[slot])

            def _issue_gather(j):
                slot = lax.rem(j, NBUF)
                pltpu.make_async_copy(
                    src4_hbm.at[pl.ds(0, CHUNK)], srcv.at[slot],
                    sem_a.at[slot]).wait()
                pltpu.make_async_copy(
                    dst_hbm.at[pl.ds(0, CHUNK)], dstv.at[slot],
                    sem_a.at[slot]).wait()
                pltpu.make_async_copy(
                    eig_hbm.at[pl.ds(0, CHUNK)], eigv.at[slot],
                    sem_a.at[slot]).wait()
                pltpu.async_copy(a4_hbm.at[srcv.at[slot]], g.at[slot],
                                 sem_b.at[slot])

            def _wait_scatters(slot):
                for buf in (g, g1, g2):
                    pltpu.make_async_copy(
                        zeros_hbm.at[pl.ds(0, CHUNK)], buf.at[slot],
                        sem_c.at[slot]).wait()

            def _compute_scatter(j):
                slot = lax.rem(j, NBUF)
                pltpu.make_async_copy(
                    zeros_hbm.at[pl.ds(0, CHUNK)], g.at[slot],
                    sem_b.at[slot]).wait()

                def _grp(i, _):
                    ev16 = eigv[slot, pl.ds(16 * i, 16)]
                    ea16 = jnp.abs(ev16)
                    for jj in range(16):
                        es = ev16[jj]
                        ea = ea16[jj]
                        k = 16 * i + jj
                        for fb in range(DH // 16):
                            sl = pl.ds(16 * fb, 16)
                            gv = g[slot, k, sl]
                            g1[slot, k, sl] = gv * ea
                            g2[slot, k, sl] = gv * es
                    return 0

                lax.fori_loop(0, CHUNK // 16, _grp, 0)

                pltpu.async_copy(g.at[slot], acc0.at[dstv.at[slot]],
                                 sem_c.at[slot], add=True)
                pltpu.async_copy(g1.at[slot], acc1.at[dstv.at[slot]],
                                 sem_c.at[slot], add=True)
                pltpu.async_copy(g2.at[slot], acc2.at[dstv.at[slot]],
                                 sem_c.at[slot], add=True)

            _load(0)
            _load(1)
            _issue_gather(0)

            def _body(ci, _):
                @pl.when(ci + 1 < n_chunks)
                def _():
                    _issue_gather(ci + 1)

                @pl.when(ci + 2 < n_chunks)
                def _():
                    @pl.when(ci >= NBUF - 2)
                    def _():
                        _wait_scatters(lax.rem(ci + 2, NBUF))
                    _load(ci + 2)

                _compute_scatter(ci)
                return 0

            lax.fori_loop(0, n_chunks, _body, 0)

            for j in range(max(0, n_chunks - NBUF), n_chunks):
                _wait_scatters(j % NBUF)

        for phase in range(2):
            _zero_accs()
            plsc.subcore_barrier()
            _run_phase(2 * phase + c)
            plsc.subcore_barrier()
            _copy_out(2 * phase + c)
            plsc.subcore_barrier()

    return sc_wide


def _make_sc_scalar(n, e):
    ehalf = e // 2
    ept = ehalf // NT
    n_chunks = ept // CHUNK2
    r0 = (n // NT) // 8 * 8
    r15 = n - (NT - 1) * r0
    f32 = jnp.float32

    @functools.partial(
        pl.kernel,
        mesh=_SC_MESH,
        compiler_params=_SC_PARAMS,
        out_type=jax.ShapeDtypeStruct((2, n, 16), f32),
        scratch_types=[
            pltpu.VMEM((NBUF2, CHUNK2), jnp.int32),
            pltpu.VMEM((NBUF2, CHUNK2), f32),
            pltpu.VMEM((NBUF2, CHUNK2, 16), f32),
            pltpu.VMEM_SHARED((n, 16), f32),
            pltpu.SemaphoreType.DMA((NBUF2,)),
            pltpu.SemaphoreType.DMA((NBUF2,)),
        ],
    )
    def sc_scalar(dst_hbm, eig_hbm, zeros_hbm, s_hbm,
                  dstv, eigv, scl, accs, sem_a, sem_c):
        c = lax.axis_index("c")
        s = lax.axis_index("s")

        row0 = pl.multiple_of(s * r0, 8)

        @pl.when(s < NT - 1)
        def _():
            sl = pl.ds(row0, r0)
            pltpu.sync_copy(zeros_hbm.at[sl], accs.at[sl])

        @pl.when(s == NT - 1)
        def _():
            sl = pl.ds((NT - 1) * r0, r15)
            pltpu.sync_copy(zeros_hbm.at[sl], accs.at[sl])

        plsc.subcore_barrier()

        lane = lax.iota(jnp.int32, 16)
        is0 = lane == 0
        is1 = lane == 1
        is2 = lane == 2
        tile_base = c * ehalf + s * ept

        def _load(j):
            slot = lax.rem(j, NBUF2)
            base = pl.multiple_of(tile_base + j * CHUNK2, 8)
            pltpu.async_copy(dst_hbm.at[pl.ds(base, CHUNK2)],
                             dstv.at[slot], sem_a.at[slot])
            pltpu.async_copy(eig_hbm.at[pl.ds(base, CHUNK2)],
                             eigv.at[slot], sem_a.at[slot])

        def _wait_scatter(slot):
            pltpu.make_async_copy(
                zeros_hbm.at[pl.ds(0, CHUNK2)], scl.at[slot],
                sem_c.at[slot]).wait()

        def _compute_scatter(j):
            slot = lax.rem(j, NBUF2)
            pltpu.make_async_copy(
                dst_hbm.at[pl.ds(0, CHUNK2)], dstv.at[slot],
                sem_a.at[slot]).wait()
            pltpu.make_async_copy(
                eig_hbm.at[pl.ds(0, CHUNK2)], eigv.at[slot],
                sem_a.at[slot]).wait()

            def _grp(i, _):
                ev16 = eigv[slot, pl.ds(16 * i, 16)]
                ea16 = jnp.abs(ev16)
                for j2 in range(16):
                    row = jnp.where(is0, jnp.float32(1.0),
                                    jnp.where(is1, ea16[j2],
                                              jnp.where(is2, ev16[j2],
                                                        jnp.float32(0.0))))
                    scl[slot, 16 * i + j2, pl.ds(0, 16)] = row
                return 0

            lax.fori_loop(0, CHUNK2 // 16, _grp, 0)
            pltpu.async_copy(scl.at[slot], accs.at[dstv.at[slot]],
                             sem_c.at[slot], add=True)

        _load(0)
        _load(1)

        def _body(ci, _):
            @pl.when(ci + 2 < n_chunks)
            def _():
                @pl.when(ci >= NBUF2 - 2)
                def _():
                    _wait_scatter(lax.rem(ci + 2, NBUF2))
                _load(ci + 2)

            _compute_scatter(ci)
            return 0

        lax.fori_loop(0, n_chunks, _body, 0)

        for j in range(max(0, n_chunks - NBUF2), n_chunks):
            _wait_scatter(j % NBUF2)

        plsc.subcore_barrier()

        @pl.when(s < NT - 1)
        def _():
            sl = pl.ds(row0, r0)
            pltpu.sync_copy(accs.at[sl], s_hbm.at[c, sl])

        @pl.when(s == NT - 1)
        def _():
            sl = pl.ds((NT - 1) * r0, r15)
            pltpu.sync_copy(accs.at[sl], s_hbm.at[c, sl])

    return sc_scalar


def _post_body(h_ref, b_ref, t0_ref, t1_ref, t2_ref, sa_ref, sb_ref, sn_ref,
               w2_ref, b1_ref, b2_ref, y_ref, s1_ref, s2_ref):
    i = pl.program_id(0)
    h = h_ref[...]
    bb = b_ref[...] + b1_ref[...]
    sc = sa_ref[...] + sb_ref[...]
    deg = sc[:, 0:1]
    asum = sc[:, 1:2]
    esum = sc[:, 2:3]
    degc = jnp.maximum(deg, 1.0)
    r = 1.0 / (asum + EPS)
    agg_mean = (t0_ref[...] + deg * bb) / degc
    agg_av = (t1_ref[...] + asum * bb) * r
    agg_dx = jnp.abs((t2_ref[...] + esum * bb) * r - (esum * r) * h)
    x = jnp.concatenate([h, agg_mean, agg_av, agg_dx], axis=1)
    y = jnp.dot(x, w2_ref[...], preferred_element_type=jnp.float32)
    y = (y + b2_ref[...]) * sn_ref[...]
    y_ref[...] = y

    @pl.when(i == 0)
    def _():
        s1_ref[...] = jnp.zeros_like(s1_ref)
        s2_ref[...] = jnp.zeros_like(s2_ref)

    s1_ref[...] += jnp.sum(y, axis=0, keepdims=True)
    s2_ref[...] += jnp.sum(y * y, axis=0, keepdims=True)


def _posttrans(h, b, t0, t1, t2, scal_a, scal_b, snorm, w2, b1r, b2r,
               row_block):
    n, d = h.shape
    grid = n // row_block
    rb = row_block
    return pl.pallas_call(
        _post_body,
        grid=(grid,),
        in_specs=[
            pl.BlockSpec((rb, d), lambda i: (i, 0)),
            pl.BlockSpec((rb, d), lambda i: (i, 0)),
            pl.BlockSpec((rb, d), lambda i: (i, 0)),
            pl.BlockSpec((rb, d), lambda i: (i, 0)),
            pl.BlockSpec((rb, d), lambda i: (i, 0)),
            pl.BlockSpec((rb, 16), lambda i: (i, 0)),
            pl.BlockSpec((rb, 16), lambda i: (i, 0)),
            pl.BlockSpec((rb, 1), lambda i: (i, 0)),
            pl.BlockSpec((4 * d, d), lambda i: (0, 0)),
            pl.BlockSpec((1, d), lambda i: (0, 0)),
            pl.BlockSpec((1, d), lambda i: (0, 0)),
        ],
        out_specs=[
            pl.BlockSpec((rb, d), lambda i: (i, 0)),
            pl.BlockSpec((1, d), lambda i: (0, 0)),
            pl.BlockSpec((1, d), lambda i: (0, 0)),
        ],
        out_shape=[
            jax.ShapeDtypeStruct((n, d), jnp.float32),
            jax.ShapeDtypeStruct((1, d), jnp.float32),
            jax.ShapeDtypeStruct((1, d), jnp.float32),
        ],
    )(h, b, t0, t1, t2, scal_a, scal_b, snorm, w2, b1r, b2r)


def _bn_body(y_ref, h_ref, s1_ref, s2_ref, g_ref, be_ref, n_ref, o_ref):
    n = n_ref[0]
    mu = s1_ref[...] / n
    var = s2_ref[...] / n - mu * mu
    inv = lax.rsqrt(var + BN_EPS)
    yn = (y_ref[...] - mu) * (inv * g_ref[...]) + be_ref[...]
    o_ref[...] = h_ref[...] + jnp.maximum(yn, 0.0)


def _bn_apply(y, h, s1, s2, gr, br, row_block):
    n, d = h.shape
    grid = n // row_block
    rb = row_block
    nvec = jnp.full((1,), float(n), dtype=jnp.float32)
    return pl.pallas_call(
        _bn_body,
        grid=(grid,),
        in_specs=[
            pl.BlockSpec((rb, d), lambda i: (i, 0)),
            pl.BlockSpec((rb, d), lambda i: (i, 0)),
            pl.BlockSpec((1, d), lambda i: (0, 0)),
            pl.BlockSpec((1, d), lambda i: (0, 0)),
            pl.BlockSpec((1, d), lambda i: (0, 0)),
            pl.BlockSpec((1, d), lambda i: (0, 0)),
            pl.BlockSpec(memory_space=pltpu.SMEM),
        ],
        out_specs=pl.BlockSpec((rb, d), lambda i: (i, 0)),
        out_shape=jax.ShapeDtypeStruct((n, d), jnp.float32),
    )(y, h, s1, s2, gr, br, nvec)


def kernel(h, edge_index, eig, snorm_n, W1, b1, W2, b2, gamma, beta):
    n, d = h.shape
    e = edge_index.shape[1]

    wcat = jnp.concatenate([W1[:d], W1[d:]], axis=1)
    p = _pretrans(h, wcat, row_block=1000)
    a = p[:, :d]
    b = p[:, d:]
    a4 = jnp.concatenate([a[:, 0:DH], a[:, DH:2 * DH],
                          a[:, 2 * DH:3 * DH], a[:, 3 * DH:]], axis=0)

    src = edge_index[0]
    dst = edge_index[1]
    src4 = jnp.concatenate([src, src + n, src + 2 * n, src + 3 * n])
    ev = eig[:, 0]

    zw = jnp.zeros((n, DH), jnp.float32)
    zs = jnp.zeros((n, 16), jnp.float32)
    t0h, t1h, t2h = _make_sc_wide(n, e)(a4, src4, dst, ev, zw)
    sh = _make_sc_scalar(n, e)(dst, ev, zs)
    t0 = jnp.concatenate([t0h[0], t0h[1], t0h[2], t0h[3]], axis=1)
    t1 = jnp.concatenate([t1h[0], t1h[1], t1h[2], t1h[3]], axis=1)
    t2 = jnp.concatenate([t2h[0], t2h[1], t2h[2], t2h[3]], axis=1)

    b1r = b1.reshape(1, d)
    b2r = b2.reshape(1, d)
    y, s1, s2 = _posttrans(h, b, t0, t1, t2, sh[0], sh[1], snorm_n, W2,
                           b1r, b2r, row_block=1000)
    return _bn_apply(y, h, s1, s2, gamma.reshape(1, d), beta.reshape(1, d),
                     row_block=1000)


# speedup vs baseline: 8.2971x; 1.0131x over previous
import functools

import jax
import jax.numpy as jnp
from jax import lax
from jax.experimental import pallas as pl
from jax.experimental.pallas import tpu as pltpu
from jax.experimental.pallas import tpu_sc as plsc

EPS = 1e-8
BN_EPS = 1e-5


def _mm_body(h_ref, w_ref, o_ref):
    o_ref[...] = jnp.dot(h_ref[...], w_ref[...],
                         preferred_element_type=jnp.float32)


def _pretrans(h, wcat, row_block):
    n, d = h.shape
    grid = n // row_block
    return pl.pallas_call(
        _mm_body,
        grid=(grid,),
        in_specs=[
            pl.BlockSpec((row_block, d), lambda i: (i, 0)),
            pl.BlockSpec((d, 2 * d), lambda i: (0, 0)),
        ],
        out_specs=pl.BlockSpec((row_block, 2 * d), lambda i: (i, 0)),
        out_shape=jax.ShapeDtypeStruct((n, 2 * d), jnp.float32),
    )(h, wcat)


NT = 16
CHUNK = 80
NBUF = 4
CHUNK2 = 80
NBUF2 = 4
DH = 32
RW = 3 * DH + 16

_SC_MESH = plsc.VectorSubcoreMesh(core_axis_name="c", subcore_axis_name="s")
_SC_PARAMS = pltpu.CompilerParams(use_tc_tiling_on_sc=False)


def _make_sc_wide(n, e):
    edges_per_tile = e // NT
    n_chunks = edges_per_tile // CHUNK
    r0 = (n // NT) // 8 * 8
    r15 = n - (NT - 1) * r0
    f32 = jnp.float32

    @functools.partial(
        pl.kernel,
        mesh=_SC_MESH,
        compiler_params=_SC_PARAMS,
        out_type=jax.ShapeDtypeStruct((4, n, RW), f32),
        scratch_types=[
            pltpu.VMEM((NBUF, CHUNK), jnp.int32),
            pltpu.VMEM((NBUF, CHUNK), jnp.int32),
            pltpu.VMEM((NBUF, CHUNK), f32),
            pltpu.VMEM((NBUF, CHUNK, DH), f32),
            pltpu.VMEM((NBUF, CHUNK, RW), f32),
            pltpu.VMEM_SHARED((n, RW), f32),
            pltpu.SemaphoreType.DMA((NBUF,)),
            pltpu.SemaphoreType.DMA((NBUF,)),
            pltpu.SemaphoreType.DMA((NBUF,)),
        ],
    )
    def sc_wide(a4_hbm, src4_hbm, dst_hbm, eig_hbm, zeros_hbm,
                t_hbm, srcv, dstv, eigv, g, gall,
                acc, sem_a, sem_b, sem_c):
        c = lax.axis_index("c")
        s = lax.axis_index("s")
        row0 = pl.multiple_of(s * r0, 8)
        tile_base = s * edges_per_tile
        lane = lax.iota(jnp.int32, 16)
        is0 = lane == 0
        is1 = lane == 1
        is2 = lane == 2

        def _zero_accs():
            @pl.when(s < NT - 1)
            def _():
                sl = pl.ds(row0, r0)
                pltpu.sync_copy(zeros_hbm.at[sl], acc.at[sl])

            @pl.when(s == NT - 1)
            def _():
                sl = pl.ds((NT - 1) * r0, r15)
                pltpu.sync_copy(zeros_hbm.at[sl], acc.at[sl])

        def _copy_out(q):
            @pl.when(s < NT - 1)
            def _():
                sl = pl.ds(row0, r0)
                pltpu.sync_copy(acc.at[sl], t_hbm.at[q, sl])

            @pl.when(s == NT - 1)
            def _():
                sl = pl.ds((NT - 1) * r0, r15)
                pltpu.sync_copy(acc.at[sl], t_hbm.at[q, sl])

        def _run_phase(q):
            def _load(j):
                slot = lax.rem(j, NBUF)
                base = tile_base + j * CHUNK
                src_off = pl.multiple_of(q * e + base, 8)
                pltpu.async_copy(src4_hbm.at[pl.ds(src_off, CHUNK)],
                                 srcv.at[slot], sem_a.at[slot])
                pltpu.async_copy(dst_hbm.at[pl.ds(base, CHUNK)],
                                 dstv.at[slot], sem_a.at[slot])
                pltpu.async_copy(eig_hbm.at[pl.ds(base, CHUNK)],
                                 eigv.at[slot], sem_a.at[slot])

            def _issue_gather(j):
                slot = lax.rem(j, NBUF)
                pltpu.make_async_copy(
                    src4_hbm.at[pl.ds(0, CHUNK)], srcv.at[slot],
                    sem_a.at[slot]).wait()
                pltpu.make_async_copy(
                    dst_hbm.at[pl.ds(0, CHUNK)], dstv.at[slot],
                    sem_a.at[slot]).wait()
                pltpu.make_async_copy(
                    eig_hbm.at[pl.ds(0, CHUNK)], eigv.at[slot],
                    sem_a.at[slot]).wait()
                pltpu.async_copy(a4_hbm.at[srcv.at[slot]], g.at[slot],
                                 sem_b.at[slot])

            def _wait_scatters(slot):
                pltpu.make_async_copy(
                    zeros_hbm.at[pl.ds(0, CHUNK)], gall.at[slot],
                    sem_c.at[slot]).wait()

            def _compute_scatter(j):
                slot = lax.rem(j, NBUF)
                pltpu.make_async_copy(
                    a4_hbm.at[pl.ds(0, CHUNK)], g.at[slot],
                    sem_b.at[slot]).wait()

                def _grp(i, _):
                    ev16 = eigv[slot, pl.ds(16 * i, 16)]
                    ea16 = jnp.abs(ev16)
                    for jj in range(16):
                        es = ev16[jj]
                        ea = ea16[jj]
                        k = 16 * i + jj
                        scal_row = jnp.where(
                            is0, jnp.float32(1.0),
                            jnp.where(is1, ea,
                                      jnp.where(is2, es, jnp.float32(0.0))))
                        gall[slot, k, pl.ds(3 * DH, 16)] = scal_row
                        for fb in range(DH // 16):
                            sl0 = pl.ds(16 * fb, 16)
                            gv = g[slot, k, sl0]
                            gall[slot, k, sl0] = gv
                            gall[slot, k, pl.ds(DH + 16 * fb, 16)] = gv * ea
                            gall[slot, k, pl.ds(2 * DH + 16 * fb, 16)] = \
                                gv * es
                    return 0

                lax.fori_loop(0, CHUNK // 16, _grp, 0)

                pltpu.async_copy(gall.at[slot], acc.at[dstv.at[slot]],
                                 sem_c.at[slot], add=True)

            _load(0)
            _load(1)
            _issue_gather(0)

            def _body(ci, _):
                @pl.when(ci + 1 < n_chunks)
                def _():
                    _issue_gather(ci + 1)

                @pl.when(ci + 2 < n_chunks)
                def _():
                    @pl.when(ci >= NBUF - 2)
                    def _():
                        _wait_scatters(lax.rem(ci + 2, NBUF))
                    _load(ci + 2)

                _compute_scatter(ci)
                return 0

            lax.fori_loop(0, n_chunks, _body, 0)

            for j in range(max(0, n_chunks - NBUF), n_chunks):
                _wait_scatters(j % NBUF)

        for phase in range(2):
            _zero_accs()
            plsc.subcore_barrier()
            _run_phase(2 * phase + c)
            plsc.subcore_barrier()
            _copy_out(2 * phase + c)
            plsc.subcore_barrier()

    return sc_wide


def _post_body(h_ref, b_ref, t0_ref, t1_ref, t2_ref, sa_ref, sn_ref,
               w2_ref, b1_ref, b2_ref, y_ref, s1_ref, s2_ref):
    i = pl.program_id(0)
    h = h_ref[...]
    bb = b_ref[...] + b1_ref[...]
    sc = sa_ref[...]
    deg = sc[:, 0:1]
    asum = sc[:, 1:2]
    esum = sc[:, 2:3]
    degc = jnp.maximum(deg, 1.0)
    r = 1.0 / (asum + EPS)
    agg_mean = (t0_ref[...] + deg * bb) / degc
    agg_av = (t1_ref[...] + asum * bb) * r
    agg_dx = jnp.abs((t2_ref[...] + esum * bb) * r - (esum * r) * h)
    x = jnp.concatenate([h, agg_mean, agg_av, agg_dx], axis=1)
    y = jnp.dot(x, w2_ref[...], preferred_element_type=jnp.float32)
    y = (y + b2_ref[...]) * sn_ref[...]
    y_ref[...] = y

    @pl.when(i == 0)
    def _():
        s1_ref[...] = jnp.zeros_like(s1_ref)
        s2_ref[...] = jnp.zeros_like(s2_ref)

    s1_ref[...] += jnp.sum(y, axis=0, keepdims=True)
    s2_ref[...] += jnp.sum(y * y, axis=0, keepdims=True)


def _posttrans(h, b, t0, t1, t2, scal_a, snorm, w2, b1r, b2r,
               row_block):
    n, d = h.shape
    grid = n // row_block
    rb = row_block
    return pl.pallas_call(
        _post_body,
        grid=(grid,),
        in_specs=[
            pl.BlockSpec((rb, d), lambda i: (i, 0)),
            pl.BlockSpec((rb, d), lambda i: (i, 0)),
            pl.BlockSpec((rb, d), lambda i: (i, 0)),
            pl.BlockSpec((rb, d), lambda i: (i, 0)),
            pl.BlockSpec((rb, d), lambda i: (i, 0)),
            pl.BlockSpec((rb, 16), lambda i: (i, 0)),
            pl.BlockSpec((rb, 1), lambda i: (i, 0)),
            pl.BlockSpec((4 * d, d), lambda i: (0, 0)),
            pl.BlockSpec((1, d), lambda i: (0, 0)),
            pl.BlockSpec((1, d), lambda i: (0, 0)),
        ],
        out_specs=[
            pl.BlockSpec((rb, d), lambda i: (i, 0)),
            pl.BlockSpec((1, d), lambda i: (0, 0)),
            pl.BlockSpec((1, d), lambda i: (0, 0)),
        ],
        out_shape=[
            jax.ShapeDtypeStruct((n, d), jnp.float32),
            jax.ShapeDtypeStruct((1, d), jnp.float32),
            jax.ShapeDtypeStruct((1, d), jnp.float32),
        ],
    )(h, b, t0, t1, t2, scal_a, snorm, w2, b1r, b2r)


def _bn_body(y_ref, h_ref, s1_ref, s2_ref, g_ref, be_ref, n_ref, o_ref):
    n = n_ref[0]
    mu = s1_ref[...] / n
    var = s2_ref[...] / n - mu * mu
    inv = lax.rsqrt(var + BN_EPS)
    yn = (y_ref[...] - mu) * (inv * g_ref[...]) + be_ref[...]
    o_ref[...] = h_ref[...] + jnp.maximum(yn, 0.0)


def _bn_apply(y, h, s1, s2, gr, br, row_block):
    n, d = h.shape
    grid = n // row_block
    rb = row_block
    nvec = jnp.full((1,), float(n), dtype=jnp.float32)
    return pl.pallas_call(
        _bn_body,
        grid=(grid,),
        in_specs=[
            pl.BlockSpec((rb, d), lambda i: (i, 0)),
            pl.BlockSpec((rb, d), lambda i: (i, 0)),
            pl.BlockSpec((1, d), lambda i: (0, 0)),
            pl.BlockSpec((1, d), lambda i: (0, 0)),
            pl.BlockSpec((1, d), lambda i: (0, 0)),
            pl.BlockSpec((1, d), lambda i: (0, 0)),
            pl.BlockSpec(memory_space=pltpu.SMEM),
        ],
        out_specs=pl.BlockSpec((rb, d), lambda i: (i, 0)),
        out_shape=jax.ShapeDtypeStruct((n, d), jnp.float32),
    )(y, h, s1, s2, gr, br, nvec)


def kernel(h, edge_index, eig, snorm_n, W1, b1, W2, b2, gamma, beta):
    n, d = h.shape
    e = edge_index.shape[1]

    wcat = jnp.concatenate([W1[:d], W1[d:]], axis=1)
    p = _pretrans(h, wcat, row_block=1000)
    a = p[:, :d]
    b = p[:, d:]
    a4 = jnp.concatenate([a[:, 0:DH], a[:, DH:2 * DH],
                          a[:, 2 * DH:3 * DH], a[:, 3 * DH:]], axis=0)

    src = edge_index[0]
    dst = edge_index[1]
    src4 = jnp.concatenate([src, src + n, src + 2 * n, src + 3 * n])
    ev = eig[:, 0]

    zw = jnp.zeros((n, RW), jnp.float32)
    th = _make_sc_wide(n, e)(a4, src4, dst, ev, zw)
    t0 = jnp.concatenate([th[q, :, 0:DH] for q in range(4)], axis=1)
    t1 = jnp.concatenate([th[q, :, DH:2 * DH] for q in range(4)], axis=1)
    t2 = jnp.concatenate([th[q, :, 2 * DH:3 * DH] for q in range(4)], axis=1)
    scal = th[0, :, 3 * DH:]

    b1r = b1.reshape(1, d)
    b2r = b2.reshape(1, d)
    y, s1, s2 = _posttrans(h, b, t0, t1, t2, scal, snorm_n, W2,
                           b1r, b2r, row_block=1000)
    return _bn_apply(y, h, s1, s2, gamma.reshape(1, d), beta.reshape(1, d),
                     row_block=1000)


# speedup vs baseline: 8.4044x; 1.0129x over previous
import functools

import jax
import jax.numpy as jnp
from jax import lax
from jax.experimental import pallas as pl
from jax.experimental.pallas import tpu as pltpu
from jax.experimental.pallas import tpu_sc as plsc

EPS = 1e-8
BN_EPS = 1e-5


def _mm_body(h_ref, w_ref, o_ref):
    o_ref[...] = jnp.dot(h_ref[...], w_ref[...],
                         preferred_element_type=jnp.float32)


def _pretrans(h, wcat, row_block):
    n, d = h.shape
    grid = n // row_block
    return pl.pallas_call(
        _mm_body,
        grid=(grid,),
        in_specs=[
            pl.BlockSpec((row_block, d), lambda i: (i, 0)),
            pl.BlockSpec((d, 2 * d), lambda i: (0, 0)),
        ],
        out_specs=pl.BlockSpec((row_block, 2 * d), lambda i: (i, 0)),
        out_shape=jax.ShapeDtypeStruct((n, 2 * d), jnp.float32),
    )(h, wcat)


NT = 16
CHUNK = 80
NBUF = 4
CHUNK2 = 80
NBUF2 = 4
DH = 32
RW = 3 * DH + 16

_SC_MESH = plsc.VectorSubcoreMesh(core_axis_name="c", subcore_axis_name="s")
_SC_PARAMS = pltpu.CompilerParams(use_tc_tiling_on_sc=False)


def _make_sc_wide(n, e):
    edges_per_tile = e // NT
    n_chunks = edges_per_tile // CHUNK
    r0 = (n // NT) // 8 * 8
    r15 = n - (NT - 1) * r0
    f32 = jnp.float32

    @functools.partial(
        pl.kernel,
        mesh=_SC_MESH,
        compiler_params=_SC_PARAMS,
        out_type=jax.ShapeDtypeStruct((4, n, RW), f32),
        scratch_types=[
            pltpu.VMEM((NBUF, CHUNK), jnp.int32),
            pltpu.VMEM((NBUF, CHUNK), jnp.int32),
            pltpu.VMEM((NBUF, CHUNK), f32),
            pltpu.VMEM((NBUF, CHUNK, DH), f32),
            pltpu.VMEM((NBUF, CHUNK, RW), f32),
            pltpu.VMEM_SHARED((n, RW), f32),
            pltpu.SemaphoreType.DMA((NBUF,)),
            pltpu.SemaphoreType.DMA((NBUF,)),
            pltpu.SemaphoreType.DMA((NBUF,)),
        ],
    )
    def sc_wide(a4_hbm, src_hbm, dst_hbm, eig_hbm, zeros_hbm,
                t_hbm, srcv, dstv, eigv, g, gall,
                acc, sem_a, sem_b, sem_c):
        c = lax.axis_index("c")
        s = lax.axis_index("s")
        row0 = pl.multiple_of(s * r0, 8)
        tile_base = s * edges_per_tile
        lane = lax.iota(jnp.int32, 16)
        is0 = lane == 0
        is1 = lane == 1
        is2 = lane == 2

        def _zero_accs():
            @pl.when(s < NT - 1)
            def _():
                sl = pl.ds(row0, r0)
                pltpu.sync_copy(zeros_hbm.at[sl], acc.at[sl])

            @pl.when(s == NT - 1)
            def _():
                sl = pl.ds((NT - 1) * r0, r15)
                pltpu.sync_copy(zeros_hbm.at[sl], acc.at[sl])

        def _copy_out(q):
            @pl.when(s < NT - 1)
            def _():
                sl = pl.ds(row0, r0)
                pltpu.sync_copy(acc.at[sl], t_hbm.at[q, sl])

            @pl.when(s == NT - 1)
            def _():
                sl = pl.ds((NT - 1) * r0, r15)
                pltpu.sync_copy(acc.at[sl], t_hbm.at[q, sl])

        def _run_phase(q):
            def _load(j):
                slot = lax.rem(j, NBUF)
                base = pl.multiple_of(tile_base + j * CHUNK, 8)
                pltpu.async_copy(src_hbm.at[pl.ds(base, CHUNK)],
                                 srcv.at[slot], sem_a.at[slot])
                pltpu.async_copy(dst_hbm.at[pl.ds(base, CHUNK)],
                                 dstv.at[slot], sem_a.at[slot])
                pltpu.async_copy(eig_hbm.at[pl.ds(base, CHUNK)],
                                 eigv.at[slot], sem_a.at[slot])

            def _issue_gather(j):
                slot = lax.rem(j, NBUF)
                pltpu.make_async_copy(
                    src_hbm.at[pl.ds(0, CHUNK)], srcv.at[slot],
                    sem_a.at[slot]).wait()
                pltpu.make_async_copy(
                    dst_hbm.at[pl.ds(0, CHUNK)], dstv.at[slot],
                    sem_a.at[slot]).wait()
                pltpu.make_async_copy(
                    eig_hbm.at[pl.ds(0, CHUNK)], eigv.at[slot],
                    sem_a.at[slot]).wait()
                off = q * n
                for i4 in range(CHUNK // 16):
                    sl4 = pl.ds(16 * i4, 16)
                    srcv[slot, sl4] = srcv[slot, sl4] + off
                pltpu.async_copy(a4_hbm.at[srcv.at[slot]], g.at[slot],
                                 sem_b.at[slot])

            def _wait_scatters(slot):
                pltpu.make_async_copy(
                    zeros_hbm.at[pl.ds(0, CHUNK)], gall.at[slot],
                    sem_c.at[slot]).wait()

            def _compute_scatter(j):
                slot = lax.rem(j, NBUF)
                pltpu.make_async_copy(
                    a4_hbm.at[pl.ds(0, CHUNK)], g.at[slot],
                    sem_b.at[slot]).wait()

                def _grp(i, _):
                    ev16 = eigv[slot, pl.ds(16 * i, 16)]
                    ea16 = jnp.abs(ev16)
                    for jj in range(16):
                        es = ev16[jj]
                        ea = ea16[jj]
                        k = 16 * i + jj
                        scal_row = jnp.where(
                            is0, jnp.float32(1.0),
                            jnp.where(is1, ea,
                                      jnp.where(is2, es, jnp.float32(0.0))))
                        gall[slot, k, pl.ds(3 * DH, 16)] = scal_row
                        for fb in range(DH // 16):
                            sl0 = pl.ds(16 * fb, 16)
                            gv = g[slot, k, sl0]
                            gall[slot, k, sl0] = gv
                            gall[slot, k, pl.ds(DH + 16 * fb, 16)] = gv * ea
                            gall[slot, k, pl.ds(2 * DH + 16 * fb, 16)] = \
                                gv * es
                    return 0

                lax.fori_loop(0, CHUNK // 16, _grp, 0)

                pltpu.async_copy(gall.at[slot], acc.at[dstv.at[slot]],
                                 sem_c.at[slot], add=True)

            _load(0)
            _load(1)
            _issue_gather(0)

            def _body(ci, _):
                @pl.when(ci + 1 < n_chunks)
                def _():
                    _issue_gather(ci + 1)

                @pl.when(ci + 2 < n_chunks)
                def _():
                    @pl.when(ci >= NBUF - 2)
                    def _():
                        _wait_scatters(lax.rem(ci + 2, NBUF))
                    _load(ci + 2)

                _compute_scatter(ci)
                return 0

            lax.fori_loop(0, n_chunks, _body, 0)

            for j in range(max(0, n_chunks - NBUF), n_chunks):
                _wait_scatters(j % NBUF)

        for phase in range(2):
            _zero_accs()
            plsc.subcore_barrier()
            _run_phase(2 * phase + c)
            plsc.subcore_barrier()
            _copy_out(2 * phase + c)
            plsc.subcore_barrier()

    return sc_wide


def _post_body(h_ref, b_ref, t0_ref, t1_ref, t2_ref, sa_ref, sn_ref,
               w2_ref, b1_ref, b2_ref, y_ref, s1_ref, s2_ref):
    i = pl.program_id(0)
    h = h_ref[...]
    bb = b_ref[...] + b1_ref[...]
    sc = sa_ref[...]
    deg = sc[:, 0:1]
    asum = sc[:, 1:2]
    esum = sc[:, 2:3]
    degc = jnp.maximum(deg, 1.0)
    r = 1.0 / (asum + EPS)
    agg_mean = (t0_ref[...] + deg * bb) / degc
    agg_av = (t1_ref[...] + asum * bb) * r
    agg_dx = jnp.abs((t2_ref[...] + esum * bb) * r - (esum * r) * h)
    x = jnp.concatenate([h, agg_mean, agg_av, agg_dx], axis=1)
    y = jnp.dot(x, w2_ref[...], preferred_element_type=jnp.float32)
    y = (y + b2_ref[...]) * sn_ref[...]
    y_ref[...] = y

    @pl.when(i == 0)
    def _():
        s1_ref[...] = jnp.zeros_like(s1_ref)
        s2_ref[...] = jnp.zeros_like(s2_ref)

    s1_ref[...] += jnp.sum(y, axis=0, keepdims=True)
    s2_ref[...] += jnp.sum(y * y, axis=0, keepdims=True)


def _posttrans(h, b, t0, t1, t2, scal_a, snorm, w2, b1r, b2r,
               row_block):
    n, d = h.shape
    grid = n // row_block
    rb = row_block
    return pl.pallas_call(
        _post_body,
        grid=(grid,),
        in_specs=[
            pl.BlockSpec((rb, d), lambda i: (i, 0)),
            pl.BlockSpec((rb, d), lambda i: (i, 0)),
            pl.BlockSpec((rb, d), lambda i: (i, 0)),
            pl.BlockSpec((rb, d), lambda i: (i, 0)),
            pl.BlockSpec((rb, d), lambda i: (i, 0)),
            pl.BlockSpec((rb, 16), lambda i: (i, 0)),
            pl.BlockSpec((rb, 1), lambda i: (i, 0)),
            pl.BlockSpec((4 * d, d), lambda i: (0, 0)),
            pl.BlockSpec((1, d), lambda i: (0, 0)),
            pl.BlockSpec((1, d), lambda i: (0, 0)),
        ],
        out_specs=[
            pl.BlockSpec((rb, d), lambda i: (i, 0)),
            pl.BlockSpec((1, d), lambda i: (0, 0)),
            pl.BlockSpec((1, d), lambda i: (0, 0)),
        ],
        out_shape=[
            jax.ShapeDtypeStruct((n, d), jnp.float32),
            jax.ShapeDtypeStruct((1, d), jnp.float32),
            jax.ShapeDtypeStruct((1, d), jnp.float32),
        ],
    )(h, b, t0, t1, t2, scal_a, snorm, w2, b1r, b2r)


def _bn_body(y_ref, h_ref, s1_ref, s2_ref, g_ref, be_ref, n_ref, o_ref):
    n = n_ref[0]
    mu = s1_ref[...] / n
    var = s2_ref[...] / n - mu * mu
    inv = lax.rsqrt(var + BN_EPS)
    yn = (y_ref[...] - mu) * (inv * g_ref[...]) + be_ref[...]
    o_ref[...] = h_ref[...] + jnp.maximum(yn, 0.0)


def _bn_apply(y, h, s1, s2, gr, br, row_block):
    n, d = h.shape
    grid = n // row_block
    rb = row_block
    nvec = jnp.full((1,), float(n), dtype=jnp.float32)
    return pl.pallas_call(
        _bn_body,
        grid=(grid,),
        in_specs=[
            pl.BlockSpec((rb, d), lambda i: (i, 0)),
            pl.BlockSpec((rb, d), lambda i: (i, 0)),
            pl.BlockSpec((1, d), lambda i: (0, 0)),
            pl.BlockSpec((1, d), lambda i: (0, 0)),
            pl.BlockSpec((1, d), lambda i: (0, 0)),
            pl.BlockSpec((1, d), lambda i: (0, 0)),
            pl.BlockSpec(memory_space=pltpu.SMEM),
        ],
        out_specs=pl.BlockSpec((rb, d), lambda i: (i, 0)),
        out_shape=jax.ShapeDtypeStruct((n, d), jnp.float32),
    )(y, h, s1, s2, gr, br, nvec)


def kernel(h, edge_index, eig, snorm_n, W1, b1, W2, b2, gamma, beta):
    n, d = h.shape
    e = edge_index.shape[1]

    wcat = jnp.concatenate([W1[:d], W1[d:]], axis=1)
    p = _pretrans(h, wcat, row_block=1000)
    a = p[:, :d]
    b = p[:, d:]
    a4 = jnp.concatenate([a[:, 0:DH], a[:, DH:2 * DH],
                          a[:, 2 * DH:3 * DH], a[:, 3 * DH:]], axis=0)

    src = edge_index[0]
    dst = edge_index[1]
    ev = eig[:, 0]

    zw = jnp.zeros((n, RW), jnp.float32)
    th = _make_sc_wide(n, e)(a4, src, dst, ev, zw)
    t0 = jnp.concatenate([th[q, :, 0:DH] for q in range(4)], axis=1)
    t1 = jnp.concatenate([th[q, :, DH:2 * DH] for q in range(4)], axis=1)
    t2 = jnp.concatenate([th[q, :, 2 * DH:3 * DH] for q in range(4)], axis=1)
    scal = th[0, :, 3 * DH:]

    b1r = b1.reshape(1, d)
    b2r = b2.reshape(1, d)
    y, s1, s2 = _posttrans(h, b, t0, t1, t2, scal, snorm_n, W2,
                           b1r, b2r, row_block=1000)
    return _bn_apply(y, h, s1, s2, gamma.reshape(1, d), beta.reshape(1, d),
                     row_block=1000)


# speedup vs baseline: 8.4046x; 1.0000x over previous
import functools

import jax
import jax.numpy as jnp
from jax import lax
from jax.experimental import pallas as pl
from jax.experimental.pallas import tpu as pltpu
from jax.experimental.pallas import tpu_sc as plsc

EPS = 1e-8
BN_EPS = 1e-5


def _mm_body(h_ref, w_ref, o_ref):
    o_ref[...] = jnp.dot(h_ref[...], w_ref[...],
                         preferred_element_type=jnp.float32)


def _pretrans(h, wcat, row_block):
    n, d = h.shape
    grid = n // row_block
    return pl.pallas_call(
        _mm_body,
        grid=(grid,),
        in_specs=[
            pl.BlockSpec((row_block, d), lambda i: (i, 0)),
            pl.BlockSpec((d, 2 * d), lambda i: (0, 0)),
        ],
        out_specs=pl.BlockSpec((row_block, 2 * d), lambda i: (i, 0)),
        out_shape=jax.ShapeDtypeStruct((n, 2 * d), jnp.float32),
    )(h, wcat)


NT = 16
CHUNK = 80
NBUF = 4
DH = 32
RW = 3 * DH + 16

_SC_MESH = plsc.VectorSubcoreMesh(core_axis_name="c", subcore_axis_name="s")
_SC_PARAMS = pltpu.CompilerParams(use_tc_tiling_on_sc=False)


def _make_sc_wide(n, e):
    edges_per_tile = e // NT
    n_chunks = edges_per_tile // CHUNK
    r0 = (n // NT) // 8 * 8
    r15 = n - (NT - 1) * r0
    f32 = jnp.float32

    @functools.partial(
        pl.kernel,
        mesh=_SC_MESH,
        compiler_params=_SC_PARAMS,
        out_type=jax.ShapeDtypeStruct((4, n, RW), f32),
        scratch_types=[
            pltpu.VMEM((NBUF, CHUNK), jnp.int32),
            pltpu.VMEM((NBUF, CHUNK), jnp.int32),
            pltpu.VMEM((NBUF, CHUNK), f32),
            pltpu.VMEM((NBUF, CHUNK, DH), f32),
            pltpu.VMEM((NBUF, CHUNK, RW), f32),
            pltpu.VMEM_SHARED((n, RW), f32),
            pltpu.SemaphoreType.DMA((NBUF,)),
            pltpu.SemaphoreType.DMA((NBUF,)),
            pltpu.SemaphoreType.DMA((NBUF,)),
        ],
    )
    def sc_wide(a4_hbm, src_hbm, dst_hbm, eig_hbm, zeros_hbm,
                t_hbm, srcv, dstv, eigv, g, gall,
                acc, sem_a, sem_b, sem_c):
        c = lax.axis_index("c")
        s = lax.axis_index("s")
        row0 = pl.multiple_of(s * r0, 8)
        tile_base = s * edges_per_tile
        lane = lax.iota(jnp.int32, 16)
        is0 = lane == 0
        is1 = lane == 1
        is2 = lane == 2

        def _zero_accs():
            @pl.when(s < NT - 1)
            def _():
                sl = pl.ds(row0, r0)
                pltpu.sync_copy(zeros_hbm.at[sl], acc.at[sl])

            @pl.when(s == NT - 1)
            def _():
                sl = pl.ds((NT - 1) * r0, r15)
                pltpu.sync_copy(zeros_hbm.at[sl], acc.at[sl])

        def _copy_out(q):
            @pl.when(s < NT - 1)
            def _():
                sl = pl.ds(row0, r0)
                pltpu.sync_copy(acc.at[sl], t_hbm.at[q, sl])

            @pl.when(s == NT - 1)
            def _():
                sl = pl.ds((NT - 1) * r0, r15)
                pltpu.sync_copy(acc.at[sl], t_hbm.at[q, sl])

        def _run_phase(q):
            def _load(j):
                slot = lax.rem(j, NBUF)
                base = pl.multiple_of(tile_base + j * CHUNK, 8)
                pltpu.async_copy(src_hbm.at[pl.ds(base, CHUNK)],
                                 srcv.at[slot], sem_a.at[slot])
                pltpu.async_copy(dst_hbm.at[pl.ds(base, CHUNK)],
                                 dstv.at[slot], sem_a.at[slot])
                pltpu.async_copy(eig_hbm.at[pl.ds(base, CHUNK)],
                                 eigv.at[slot], sem_a.at[slot])

            def _issue_gather(j):
                slot = lax.rem(j, NBUF)
                pltpu.make_async_copy(
                    src_hbm.at[pl.ds(0, CHUNK)], srcv.at[slot],
                    sem_a.at[slot]).wait()
                pltpu.make_async_copy(
                    dst_hbm.at[pl.ds(0, CHUNK)], dstv.at[slot],
                    sem_a.at[slot]).wait()
                pltpu.make_async_copy(
                    eig_hbm.at[pl.ds(0, CHUNK)], eigv.at[slot],
                    sem_a.at[slot]).wait()
                off = q * n
                for i4 in range(CHUNK // 16):
                    sl4 = pl.ds(16 * i4, 16)
                    srcv[slot, sl4] = srcv[slot, sl4] + off
                pltpu.async_copy(a4_hbm.at[srcv.at[slot]], g.at[slot],
                                 sem_b.at[slot])

            def _wait_scatters(slot):
                pltpu.make_async_copy(
                    zeros_hbm.at[pl.ds(0, CHUNK)], gall.at[slot],
                    sem_c.at[slot]).wait()

            def _compute_scatter(j):
                slot = lax.rem(j, NBUF)
                pltpu.make_async_copy(
                    a4_hbm.at[pl.ds(0, CHUNK)], g.at[slot],
                    sem_b.at[slot]).wait()

                def _grp(i, _):
                    ev16 = eigv[slot, pl.ds(16 * i, 16)]
                    ea16 = jnp.abs(ev16)
                    for jj in range(16):
                        es = ev16[jj]
                        ea = ea16[jj]
                        k = 16 * i + jj
                        scal_row = jnp.where(
                            is0, jnp.float32(1.0),
                            jnp.where(is1, ea,
                                      jnp.where(is2, es, jnp.float32(0.0))))
                        gall[slot, k, pl.ds(3 * DH, 16)] = scal_row
                        for fb in range(DH // 16):
                            sl0 = pl.ds(16 * fb, 16)
                            gv = g[slot, k, sl0]
                            gall[slot, k, sl0] = gv
                            gall[slot, k, pl.ds(DH + 16 * fb, 16)] = gv * ea
                            gall[slot, k, pl.ds(2 * DH + 16 * fb, 16)] = \
                                gv * es
                    return 0

                lax.fori_loop(0, CHUNK // 16, _grp, 0)

                pltpu.async_copy(gall.at[slot], acc.at[dstv.at[slot]],
                                 sem_c.at[slot], add=True)

            _load(0)
            _load(1)
            _issue_gather(0)

            def _body(ci, _):
                @pl.when(ci + 1 < n_chunks)
                def _():
                    _issue_gather(ci + 1)

                @pl.when(ci + 2 < n_chunks)
                def _():
                    @pl.when(ci >= NBUF - 2)
                    def _():
                        _wait_scatters(lax.rem(ci + 2, NBUF))
                    _load(ci + 2)

                _compute_scatter(ci)
                return 0

            lax.fori_loop(0, n_chunks, _body, 0)

            for j in range(max(0, n_chunks - NBUF), n_chunks):
                _wait_scatters(j % NBUF)

        for phase in range(2):
            _zero_accs()
            plsc.subcore_barrier()
            _run_phase(2 * phase + c)
            plsc.subcore_barrier()
            _copy_out(2 * phase + c)
            plsc.subcore_barrier()

    return sc_wide


def _post_body(h_ref, b_ref, t0_ref, t1_ref, t2_ref, sa_ref, sn_ref,
               w2_ref, b1_ref, b2_ref, y_ref, s1_ref, s2_ref):
    i = pl.program_id(0)
    h = h_ref[...]
    bb = b_ref[...] + b1_ref[...]
    sc = sa_ref[...]
    deg = sc[:, 0:1]
    asum = sc[:, 1:2]
    esum = sc[:, 2:3]
    degc = jnp.maximum(deg, 1.0)
    r = 1.0 / (asum + EPS)
    agg_mean = (t0_ref[...] + deg * bb) / degc
    agg_av = (t1_ref[...] + asum * bb) * r
    agg_dx = jnp.abs((t2_ref[...] + esum * bb) * r - (esum * r) * h)
    x = jnp.concatenate([h, agg_mean, agg_av, agg_dx], axis=1)
    y = jnp.dot(x, w2_ref[...], preferred_element_type=jnp.float32)
    y = (y + b2_ref[...]) * sn_ref[...]
    y_ref[...] = y

    @pl.when(i == 0)
    def _():
        s1_ref[...] = jnp.zeros_like(s1_ref)
        s2_ref[...] = jnp.zeros_like(s2_ref)

    s1_ref[...] += jnp.sum(y, axis=0, keepdims=True)
    s2_ref[...] += jnp.sum(y * y, axis=0, keepdims=True)


def _posttrans(h, b, t0, t1, t2, scal_a, snorm, w2, b1r, b2r,
               row_block):
    n, d = h.shape
    grid = n // row_block
    rb = row_block
    return pl.pallas_call(
        _post_body,
        grid=(grid,),
        in_specs=[
            pl.BlockSpec((rb, d), lambda i: (i, 0)),
            pl.BlockSpec((rb, d), lambda i: (i, 0)),
            pl.BlockSpec((rb, d), lambda i: (i, 0)),
            pl.BlockSpec((rb, d), lambda i: (i, 0)),
            pl.BlockSpec((rb, d), lambda i: (i, 0)),
            pl.BlockSpec((rb, 16), lambda i: (i, 0)),
            pl.BlockSpec((rb, 1), lambda i: (i, 0)),
            pl.BlockSpec((4 * d, d), lambda i: (0, 0)),
            pl.BlockSpec((1, d), lambda i: (0, 0)),
            pl.BlockSpec((1, d), lambda i: (0, 0)),
        ],
        out_specs=[
            pl.BlockSpec((rb, d), lambda i: (i, 0)),
            pl.BlockSpec((1, d), lambda i: (0, 0)),
            pl.BlockSpec((1, d), lambda i: (0, 0)),
        ],
        out_shape=[
            jax.ShapeDtypeStruct((n, d), jnp.float32),
            jax.ShapeDtypeStruct((1, d), jnp.float32),
            jax.ShapeDtypeStruct((1, d), jnp.float32),
        ],
    )(h, b, t0, t1, t2, scal_a, snorm, w2, b1r, b2r)


def _bn_body(y_ref, h_ref, s1_ref, s2_ref, g_ref, be_ref, n_ref, o_ref):
    n = n_ref[0]
    mu = s1_ref[...] / n
    var = s2_ref[...] / n - mu * mu
    inv = lax.rsqrt(var + BN_EPS)
    yn = (y_ref[...] - mu) * (inv * g_ref[...]) + be_ref[...]
    o_ref[...] = h_ref[...] + jnp.maximum(yn, 0.0)


def _bn_apply(y, h, s1, s2, gr, br, row_block):
    n, d = h.shape
    grid = n // row_block
    rb = row_block
    nvec = jnp.full((1,), float(n), dtype=jnp.float32)
    return pl.pallas_call(
        _bn_body,
        grid=(grid,),
        in_specs=[
            pl.BlockSpec((rb, d), lambda i: (i, 0)),
            pl.BlockSpec((rb, d), lambda i: (i, 0)),
            pl.BlockSpec((1, d), lambda i: (0, 0)),
            pl.BlockSpec((1, d), lambda i: (0, 0)),
            pl.BlockSpec((1, d), lambda i: (0, 0)),
            pl.BlockSpec((1, d), lambda i: (0, 0)),
            pl.BlockSpec(memory_space=pltpu.SMEM),
        ],
        out_specs=pl.BlockSpec((rb, d), lambda i: (i, 0)),
        out_shape=jax.ShapeDtypeStruct((n, d), jnp.float32),
    )(y, h, s1, s2, gr, br, nvec)


def kernel(h, edge_index, eig, snorm_n, W1, b1, W2, b2, gamma, beta):
    n, d = h.shape
    e = edge_index.shape[1]

    wcat = jnp.concatenate([W1[:d], W1[d:]], axis=1)
    p = _pretrans(h, wcat, row_block=1000)
    a = p[:, :d]
    b = p[:, d:]
    a4 = jnp.concatenate([a[:, 0:DH], a[:, DH:2 * DH],
                          a[:, 2 * DH:3 * DH], a[:, 3 * DH:]], axis=0)

    src = edge_index[0]
    dst = edge_index[1]
    ev = eig[:, 0]

    zw = jnp.zeros((n, RW), jnp.float32)
    th = _make_sc_wide(n, e)(a4, src, dst, ev, zw)
    t0 = jnp.concatenate([th[q, :, 0:DH] for q in range(4)], axis=1)
    t1 = jnp.concatenate([th[q, :, DH:2 * DH] for q in range(4)], axis=1)
    t2 = jnp.concatenate([th[q, :, 2 * DH:3 * DH] for q in range(4)], axis=1)
    scal = th[0, :, 3 * DH:]

    b1r = b1.reshape(1, d)
    b2r = b2.reshape(1, d)
    y, s1, s2 = _posttrans(h, b, t0, t1, t2, scal, snorm_n, W2,
                           b1r, b2r, row_block=1000)
    return _bn_apply(y, h, s1, s2, gamma.reshape(1, d), beta.reshape(1, d),
                     row_block=1000)
